# baseline (speedup 1.0000x reference)
import functools

import jax
import jax.numpy as jnp
from jax.experimental import pallas as pl
from jax.experimental.pallas import tpu as pltpu

_N = 10000
_BLK = 400


def _pass1_body(x_ref, W1_ref, b1_ref, adj_ref, h_ref, u_ref):
    @pl.when(pl.program_id(0) == 0)
    def _():
        u_ref[...] = jnp.dot(x_ref[...], W1_ref[...],
                             preferred_element_type=jnp.float32)

    acc = jnp.dot(adj_ref[...], u_ref[...], preferred_element_type=jnp.float32)
    h_ref[...] = jnp.maximum(acc + b1_ref[...], 0.0)


def _pass2_body(h_ref, W2_ref, b2_ref, W3_ref, b3_ref, adj_ref, out_ref,
                g_ref, m_ref):
    @pl.when(pl.program_id(0) == 0)
    def _():
        g_ref[...] = jnp.dot(h_ref[...], W2_ref[...],
                             preferred_element_type=jnp.float32)
        m_ref[...] = jnp.full_like(m_ref, -jnp.inf)

    part = jnp.dot(adj_ref[...], g_ref[...],
                   preferred_element_type=jnp.float32) + b2_ref[...]
    m_ref[...] = jnp.maximum(m_ref[...], jnp.max(part, axis=0, keepdims=True))

    @pl.when(pl.program_id(0) == pl.num_programs(0) - 1)
    def _():
        out_ref[...] = jnp.dot(m_ref[...], W3_ref[...],
                               preferred_element_type=jnp.float32) + b3_ref[...]


@jax.jit
def kernel(x, adj, W1, b1, W2, b2, W3, b3):
    n, nfeat = x.shape
    nhid = W1.shape[1]
    nout = W2.shape[1]
    nblocks = n // _BLK

    b1r = b1.reshape(1, nhid)
    b2r = b2.reshape(1, nout)
    b3r = b3.reshape(1, 1)

    h = pl.pallas_call(
        _pass1_body,
        grid=(nblocks,),
        in_specs=[
            pl.BlockSpec((n, nfeat), lambda i: (0, 0)),
            pl.BlockSpec((nfeat, nhid), lambda i: (0, 0)),
            pl.BlockSpec((1, nhid), lambda i: (0, 0)),
            pl.BlockSpec((_BLK, n), lambda i: (i, 0)),
        ],
        out_specs=pl.BlockSpec((_BLK, nhid), lambda i: (i, 0)),
        out_shape=jax.ShapeDtypeStruct((n, nhid), jnp.float32),
        scratch_shapes=[pltpu.VMEM((n, nhid), jnp.float32)],
    )(x, W1, b1r, adj)

    out = pl.pallas_call(
        _pass2_body,
        grid=(nblocks,),
        in_specs=[
            pl.BlockSpec((n, nhid), lambda i: (0, 0)),
            pl.BlockSpec((nhid, nout), lambda i: (0, 0)),
            pl.BlockSpec((1, nout), lambda i: (0, 0)),
            pl.BlockSpec((nout, 1), lambda i: (0, 0)),
            pl.BlockSpec((1, 1), lambda i: (0, 0)),
            pl.BlockSpec((_BLK, n), lambda i: (i, 0)),
        ],
        out_specs=pl.BlockSpec((1, 1), lambda i: (0, 0)),
        out_shape=jax.ShapeDtypeStruct((1, 1), jnp.float32),
        scratch_shapes=[
            pltpu.VMEM((n, nout), jnp.float32),
            pltpu.VMEM((1, nout), jnp.float32),
        ],
    )(h, W2, b2r, W3, b3r, adj)

    return out.reshape(1, 1, 1)


# speedup vs baseline: 1.0296x; 1.0296x over previous
import jax
import jax.numpy as jnp
from jax.experimental import pallas as pl
from jax.experimental.pallas import tpu as pltpu

_N = 10000
_BLK = 400
_NB = _N // _BLK


def _body(x_ref, W1_ref, b1_ref, W2_ref, b2_ref, W3_ref, b3_ref, adj_ref,
          out_ref, u_ref, h_ref, g_ref, m_ref):
    i = pl.program_id(0)

    @pl.when(i == 0)
    def _():
        u_ref[...] = jnp.dot(x_ref[...], W1_ref[...],
                             preferred_element_type=jnp.float32)

    @pl.when(i < _NB)
    def _():
        acc = jnp.dot(adj_ref[...], u_ref[...],
                      preferred_element_type=jnp.float32)
        h_ref[pl.ds(i * _BLK, _BLK), :] = jnp.maximum(acc + b1_ref[...], 0.0)

    @pl.when(i == _NB)
    def _():
        g_ref[...] = jnp.dot(h_ref[...], W2_ref[...],
                             preferred_element_type=jnp.float32)
        m_ref[...] = jnp.full_like(m_ref, -jnp.inf)

    @pl.when(i >= _NB)
    def _():
        part = jnp.dot(adj_ref[...], g_ref[...],
                       preferred_element_type=jnp.float32) + b2_ref[...]
        m_ref[...] = jnp.maximum(m_ref[...],
                                 jnp.max(part, axis=0, keepdims=True))

    @pl.when(i == 2 * _NB - 1)
    def _():
        out_ref[...] = jnp.dot(m_ref[...], W3_ref[...],
                               preferred_element_type=jnp.float32) + b3_ref[...]


@jax.jit
def kernel(x, adj, W1, b1, W2, b2, W3, b3):
    n, nfeat = x.shape
    nhid = W1.shape[1]
    nout = W2.shape[1]

    b1r = b1.reshape(1, nhid)
    b2r = b2.reshape(1, nout)
    b3r = b3.reshape(1, 1)

    def adj_idx(i):
        return (jnp.where(i < _NB, i, 2 * _NB - 1 - i), 0)

    out = pl.pallas_call(
        _body,
        grid=(2 * _NB,),
        in_specs=[
            pl.BlockSpec((n, nfeat), lambda i: (0, 0)),
            pl.BlockSpec((nfeat, nhid), lambda i: (0, 0)),
            pl.BlockSpec((1, nhid), lambda i: (0, 0)),
            pl.BlockSpec((nhid, nout), lambda i: (0, 0)),
            pl.BlockSpec((1, nout), lambda i: (0, 0)),
            pl.BlockSpec((nout, 1), lambda i: (0, 0)),
            pl.BlockSpec((1, 1), lambda i: (0, 0)),
            pl.BlockSpec((_BLK, n), adj_idx),
        ],
        out_specs=pl.BlockSpec((1, 1), lambda i: (0, 0)),
        out_shape=jax.ShapeDtypeStruct((1, 1), jnp.float32),
        scratch_shapes=[
            pltpu.VMEM((n, nhid), jnp.float32),
            pltpu.VMEM((n, nhid), jnp.float32),
            pltpu.VMEM((n, nout), jnp.float32),
            pltpu.VMEM((1, nout), jnp.float32),
        ],
    )(x, W1, b1r, W2, b2r, W3, b3r, adj)

    return out.reshape(1, 1, 1)
